# Initial kernel scaffold; baseline (speedup 1.0000x reference)
import functools
import jax
import jax.numpy as jnp
from jax import lax
from jax.experimental import pallas as pl
from jax.experimental.pallas import tpu as pltpu, tpu_sc as plsc

N_NODES = 50000

_mesh = plsc.VectorSubcoreMesh(core_axis_name="c", subcore_axis_name="s")


@functools.partial(
    pl.kernel,
    out_type=jax.ShapeDtypeStruct((2, 16, 16), jnp.float32),
    mesh=_mesh,
    scratch_types=[pltpu.VMEM((16,), jnp.float32)],
)
def _probe(out_hbm, acc_v):
    c = lax.axis_index("c")
    s = lax.axis_index("s")
    acc_v[...] = jnp.zeros((16,), jnp.float32)
    idx = lax.iota(jnp.int32, 16) // 4
    vals = jnp.ones((16,), jnp.float32)
    plsc.addupdate_scatter(acc_v, [idx], vals)
    pltpu.sync_copy(acc_v, out_hbm.at[c, s])


def _gcn(x, src, dst, ew, W, b):
    loop = jnp.arange(N_NODES)
    src_f = jnp.concatenate([src, loop])
    dst_f = jnp.concatenate([dst, loop])
    ew_f = jnp.concatenate([ew, jnp.ones((N_NODES,), dtype=ew.dtype)])
    deg = jnp.zeros((N_NODES,), dtype=ew.dtype).at[dst_f].add(ew_f)
    dinv = jnp.where(deg > 0, 1.0 / jnp.sqrt(deg), 0.0)
    norm = dinv[src_f] * ew_f * dinv[dst_f]
    h = x @ W
    msg = h[src_f] * norm[:, None]
    out = jnp.zeros((N_NODES, h.shape[1]), dtype=h.dtype).at[dst_f].add(msg)
    return out + b


def kernel(x, edge, edge_weight, W1, b1, W2, b2, ln_w, ln_b, Wl, bl):
    src, dst = edge[0], edge[1]
    x_t1 = _gcn(x, src, dst, edge_weight, W1, b1)
    mu = jnp.mean(x_t1)
    var = jnp.mean((x_t1 - mu) ** 2)
    x_ln = (x_t1 - mu) / jnp.sqrt(var + 1e-5) * ln_w + ln_b
    x_t2 = _gcn(x_ln, src, dst, edge_weight, W2, b2)
    out = x_t2 @ Wl.T + bl

    p = _probe()
    err = jnp.sum(jnp.abs(p[:, :, :4] - 4.0)) + jnp.sum(jnp.abs(p[:, :, 4:]))
    return out + err


# speedup vs baseline: 69.8537x; 69.8537x over previous
import functools
import jax
import jax.numpy as jnp
from jax import lax
from jax.experimental import pallas as pl
from jax.experimental.pallas import tpu as pltpu, tpu_sc as plsc

N = 50000
E = 1600000
NC, NS, NW = 2, 16, 32
EPT = E // NW
CHA = 10000
EPT_B = E // 4
CHB = 10000

_mesh = plsc.VectorSubcoreMesh(core_axis_name="c", subcore_axis_name="s")
_sc_params = pltpu.CompilerParams(needs_layout_passes=False)

def _zero(ref, n):
    z16 = jnp.zeros((16,), jnp.float32)

    def zb(i, carry):
        ref[pl.ds(i * 16, 16)] = z16
        return carry
    lax.fori_loop(0, n // 16, zb, 0)


@functools.partial(
    pl.kernel,
    out_type=jax.ShapeDtypeStruct((NW * N,), jnp.float32),
    mesh=_mesh,
    compiler_params=_sc_params,
    scratch_types=[
        pltpu.VMEM((N,), jnp.float32),
        pltpu.VMEM((CHA,), jnp.int32),
        pltpu.VMEM((CHA,), jnp.float32),
    ],
)
def _sc_deg(dst_hbm, ew_hbm, out_hbm, acc_v, dst_v, ew_v):
    c = lax.axis_index("c")
    s = lax.axis_index("s")
    wid = s * NC + c
    base = wid * EPT
    _zero(acc_v, N)

    def chunk(ci, carry):
        off = base + ci * CHA
        pltpu.sync_copy(dst_hbm.at[pl.ds(off, CHA)], dst_v)
        pltpu.sync_copy(ew_hbm.at[pl.ds(off, CHA)], ew_v)

        def grp(g, cc):
            d16 = dst_v[pl.ds(g * 16, 16)]
            w16 = ew_v[pl.ds(g * 16, 16)]
            plsc.addupdate_scatter(acc_v, [d16], w16)
            return cc

        lax.fori_loop(0, CHA // 16, grp, 0)
        return carry

    lax.fori_loop(0, EPT // CHA, chunk, 0)
    pltpu.sync_copy(acc_v, out_hbm.at[pl.ds(wid * N, N)])


@functools.partial(
    pl.kernel,
    out_type=jax.ShapeDtypeStruct((NW * N,), jnp.float32),
    mesh=_mesh,
    compiler_params=_sc_params,
    scratch_types=[
        pltpu.VMEM((N,), jnp.float32),
        pltpu.VMEM((N,), jnp.float32),
        pltpu.VMEM((CHB,), jnp.int32),
        pltpu.VMEM((CHB,), jnp.int32),
        pltpu.VMEM((CHB,), jnp.float32),
    ],
)
def _sc_conv1(src_hbm, dst_hbm, ew_hbm, xcols_hbm, out_hbm,
              col_v, acc_v, src_v, dst_v, ew_v):
    c = lax.axis_index("c")
    s = lax.axis_index("s")
    k = lax.bitwise_and(s, 7)
    q = c * 2 + lax.shift_right_logical(s, 3)
    base = q * EPT_B
    pltpu.sync_copy(xcols_hbm.at[pl.ds(k * N, N)], col_v)
    _zero(acc_v, N)

    def chunk(ci, carry):
        off = base + ci * CHB
        pltpu.sync_copy(src_hbm.at[pl.ds(off, CHB)], src_v)
        pltpu.sync_copy(dst_hbm.at[pl.ds(off, CHB)], dst_v)
        pltpu.sync_copy(ew_hbm.at[pl.ds(off, CHB)], ew_v)

        def grp(g, cc):
            s16 = src_v[pl.ds(g * 16, 16)]
            d16 = dst_v[pl.ds(g * 16, 16)]
            w16 = ew_v[pl.ds(g * 16, 16)]
            vals = plsc.load_gather(col_v, [s16]) * w16
            plsc.addupdate_scatter(acc_v, [d16], vals)
            return cc

        lax.fori_loop(0, CHB // 16, grp, 0)
        return carry

    lax.fori_loop(0, EPT_B // CHB, chunk, 0)
    r = k * 4 + q
    pltpu.sync_copy(acc_v, out_hbm.at[pl.ds(r * N, N)])


@functools.partial(
    pl.kernel,
    out_type=jax.ShapeDtypeStruct((NW * N,), jnp.float32),
    mesh=_mesh,
    compiler_params=_sc_params,
    scratch_types=[
        pltpu.VMEM((N,), jnp.float32),
        pltpu.VMEM((N,), jnp.float32),
        pltpu.VMEM((CHA,), jnp.int32),
        pltpu.VMEM((CHA,), jnp.int32),
        pltpu.VMEM((CHA,), jnp.float32),
    ],
)
def _sc_conv2(src_hbm, dst_hbm, ew_hbm, u_hbm, out_hbm,
              u_v, acc_v, src_v, dst_v, ew_v):
    c = lax.axis_index("c")
    s = lax.axis_index("s")
    wid = s * NC + c
    base = wid * EPT
    pltpu.sync_copy(u_hbm, u_v)
    _zero(acc_v, N)

    def chunk(ci, carry):
        off = base + ci * CHA
        pltpu.sync_copy(src_hbm.at[pl.ds(off, CHA)], src_v)
        pltpu.sync_copy(dst_hbm.at[pl.ds(off, CHA)], dst_v)
        pltpu.sync_copy(ew_hbm.at[pl.ds(off, CHA)], ew_v)

        def grp(g, cc):
            s16 = src_v[pl.ds(g * 16, 16)]
            d16 = dst_v[pl.ds(g * 16, 16)]
            w16 = ew_v[pl.ds(g * 16, 16)]
            vals = plsc.load_gather(u_v, [s16]) * w16
            plsc.addupdate_scatter(acc_v, [d16], vals)
            return cc

        lax.fori_loop(0, CHA // 16, grp, 0)
        return carry

    lax.fori_loop(0, EPT // CHA, chunk, 0)
    pltpu.sync_copy(acc_v, out_hbm.at[pl.ds(wid * N, N)])


def _tc1_body(degp_ref, xT_ref, xcols_ref, dinv_ref):
    deg = 1.0 + jnp.sum(degp_ref[...], axis=0)
    dinv = 1.0 / jnp.sqrt(deg)
    dinv_ref[...] = dinv
    xcols_ref[...] = xT_ref[...] * dinv[None, :]


def _tc1(degp, xT):
    return pl.pallas_call(
        _tc1_body,
        out_shape=(
            jax.ShapeDtypeStruct((8, N), jnp.float32),
            jax.ShapeDtypeStruct((N,), jnp.float32),
        ),
    )(degp, xT)


def _tc2_body(yp_ref, xT_ref, dinv_ref, W1_ref, b1_ref, lnwT_ref, lnbT_ref,
              W2_ref, Wl_ref, b2_ref, bl_ref, u_ref, v_ref):
    dinv = dinv_ref[...]
    yT = jnp.sum(yp_ref[...], axis=1)
    tT = dinv[None, :] * yT + (dinv * dinv)[None, :] * xT_ref[...]
    x1T = lax.dot_general(W1_ref[...], tT, (((0,), (0,)), ((), ())),
                          precision=lax.Precision.HIGHEST,
                          preferred_element_type=jnp.float32) + b1_ref[...][:, None]
    mu = jnp.mean(x1T)
    var = jnp.mean((x1T - mu) ** 2)
    rs = 1.0 / jnp.sqrt(var + 1e-5)
    xlnT = (x1T - mu) * rs * lnwT_ref[...] + lnbT_ref[...]
    wl_row = Wl_ref[...][0]
    w2l = jnp.sum(W2_ref[...] * wl_row[None, :], axis=1)
    z = jnp.sum(w2l[:, None] * xlnT, axis=0)
    c2 = jnp.sum(b2_ref[...] * wl_row) + bl_ref[...][0]
    u_ref[...] = dinv * z
    v_ref[...] = dinv * dinv * z + c2


def _tc2(yp, xT, dinv, W1, b1, lnwT, lnbT, W2, Wl, b2, bl):
    return pl.pallas_call(
        _tc2_body,
        out_shape=(
            jax.ShapeDtypeStruct((N,), jnp.float32),
            jax.ShapeDtypeStruct((N,), jnp.float32),
        ),
    )(yp, xT, dinv, W1, b1, lnwT, lnbT, W2, Wl, b2, bl)


def _tc3_body(gp_ref, dinv_ref, v_ref, out_ref):
    g = jnp.sum(gp_ref[...], axis=0)
    out_ref[...] = dinv_ref[...] * g + v_ref[...]


def _tc3(gp, dinv, v):
    return pl.pallas_call(
        _tc3_body,
        out_shape=jax.ShapeDtypeStruct((N,), jnp.float32),
    )(gp, dinv, v)


def kernel(x, edge, edge_weight, W1, b1, W2, b2, ln_w, ln_b, Wl, bl):
    src = edge[0].astype(jnp.int32)
    dst = edge[1].astype(jnp.int32)
    ew = edge_weight
    xT = x.T
    lnwT = ln_w.T
    lnbT = ln_b.T

    degp = _sc_deg(dst, ew).reshape(NW, N)
    xcols, dinv = _tc1(degp, xT)
    yp = _sc_conv1(src, dst, ew, xcols.reshape(-1)).reshape(8, 4, N)
    u, v = _tc2(yp, xT, dinv, W1, b1, lnwT, lnbT, W2, Wl, b2, bl)
    gp = _sc_conv2(src, dst, ew, u).reshape(NW, N)
    out = _tc3(gp, dinv, v)
    return out.reshape(N, 1)
